# baseline (speedup 1.0000x reference)
import functools
import jax
import jax.numpy as jnp
from jax import lax
from jax.experimental import pallas as pl
from jax.experimental.pallas import tpu as pltpu
from jax.experimental.pallas import tpu_sc as plsc

_B, _S, _D, _L = 128, 512, 1024, 256
_H = _D // 2
_BG = _B // 16


def _fa(a, b, c):
    t = a ^ b
    return t ^ c, (a & b) | (c & t)


def _ha(a, b):
    return a ^ b, a & b


def _count16(xs):
    c2 = []
    n1 = []
    for i in range(5):
        s, c = _fa(xs[3 * i], xs[3 * i + 1], xs[3 * i + 2])
        n1.append(s)
        c2.append(c)
    n1.append(xs[15])
    n2 = []
    for i in range(2):
        s, c = _fa(n1[3 * i], n1[3 * i + 1], n1[3 * i + 2])
        n2.append(s)
        c2.append(c)
    w0, c = _ha(n2[0], n2[1])
    c2.append(c)
    c4 = []
    m1 = []
    for i in range(2):
        s, c = _fa(c2[3 * i], c2[3 * i + 1], c2[3 * i + 2])
        m1.append(s)
        c4.append(c)
    m1.append(c2[6])
    m1.append(c2[7])
    s, c = _fa(m1[0], m1[1], m1[2])
    c4.append(c)
    w1, c = _ha(s, m1[3])
    c4.append(c)
    c8 = []
    s, c = _fa(c4[0], c4[1], c4[2])
    c8.append(c)
    w2, c = _ha(s, c4[3])
    c8.append(c)
    w3, w4 = _ha(c8[0], c8[1])
    return (w0, w1, w2, w3, w4)


def _global_add(g, w):
    g = list(g)
    g[0], carry = _ha(g[0], w[0])
    for k in range(1, 5):
        t = g[k] ^ w[k]
        s = t ^ carry
        carry = (g[k] & w[k]) | (carry & t)
        g[k] = s
    for k in range(5, 10):
        g[k], carry = _ha(g[k], carry)
    return tuple(g)


def _sc_body(x_hbm, pos_hbm, lvl_hbm, out_hbm,
             pkl_v, pkp_v, x_v, idx_v, lvlp_v, posp_v, lvl_loc, pos_loc,
             out_v, shared_v, idx_s):
    half = lax.axis_index("c")
    sid = lax.axis_index("s")

    lro = pl.multiple_of((half * _L + sid * 16) * _H, 512)
    pltpu.sync_copy(lvl_hbm.at[pl.ds(lro, 16 * _H)], pkl_v)
    pro = pl.multiple_of((half * _S + sid * 32) * _H, 512)
    pltpu.sync_copy(pos_hbm.at[pl.ds(pro, 32 * _H)], pkp_v)

    def pack_lvl(r, c_):
        word = jnp.zeros((16,), jnp.int32)
        for bb in range(32):
            off = pl.multiple_of(r * _H + bb * 16, 16)
            vi = pkl_v[pl.ds(off, 16)]
            bit = lax.shift_right_logical(vi, 31)
            word = word | (bit << bb)
        lvl_loc[pl.ds(pl.multiple_of(r * 16, 16), 16)] = word
        return c_

    lax.fori_loop(0, 16, pack_lvl, 0)

    def pack_pos(r, c_):
        word = jnp.zeros((16,), jnp.int32)
        for bb in range(32):
            off = pl.multiple_of(r * _H + bb * 16, 16)
            vi = pkp_v[pl.ds(off, 16)]
            bit = lax.shift_right_logical(vi, 31) ^ 1
            word = word | (bit << bb)
        pos_loc[pl.ds(pl.multiple_of(r * 16, 16), 16)] = word
        return c_

    lax.fori_loop(0, 32, pack_pos, 0)

    pltpu.sync_copy(lvl_loc, shared_v.at[pl.ds(pl.multiple_of(sid * 256, 256), 256)])
    pltpu.sync_copy(pos_loc, shared_v.at[pl.ds(pl.multiple_of(_L * 16 + sid * 512, 512), 512)])
    plsc.subcore_barrier()
    pltpu.sync_copy(shared_v.at[pl.ds(0, _L * 16)], lvlp_v)
    pltpu.sync_copy(shared_v.at[pl.ds(_L * 16, _S * 16)], posp_v)

    pltpu.sync_copy(x_hbm.at[pl.ds(pl.multiple_of(sid * _BG * _S, 512), _BG * _S)], x_v)

    def b_step(b, carry):
        def q_step(j, c_):
            v = x_v[pl.ds(pl.multiple_of(b * _S + j * 16, 16), 16)]
            q = v * jnp.float32(_L - 1) + jnp.float32(0.5)
            qi = jnp.clip(q.astype(jnp.int32), 0, _L - 1) * 16
            idx_v[pl.ds(j * 16, 16)] = qi
            return c_

        lax.fori_loop(0, _S // 16, q_step, 0, unroll=4)

        z = jnp.zeros((16,), jnp.int32)

        @plsc.parallel_loop(0, _S // 16, carry=(z,) * 10)
        def s_loop(t, g):
            base = t * 16
            ivec = idx_v[pl.ds(base, 16)]
            xs = []
            for j in range(16):
                io = pl.multiple_of(ivec[j], 16)
                po = pl.multiple_of((base + j) * 16, 16)
                xs.append(lvlp_v[pl.ds(io, 16)] ^ posp_v[pl.ds(po, 16)])
            return _global_add(g, _count16(xs))

        g = s_loop

        low = g[0]
        for k in range(1, 8):
            low = low | g[k]
        ge = g[9] | (g[8] & low)
        for bb in range(32):
            bit = lax.shift_right_logical(ge, bb) & 1
            valf = jnp.where(bit == 1, jnp.float32(1.0), jnp.float32(-1.0))
            out_v[pl.ds(pl.multiple_of(b * _H + bb * 16, 16), 16)] = valf
        return carry

    lax.fori_loop(0, _BG, b_step, 0)

    oo = pl.multiple_of((half * 16 + sid) * (_BG * _H), 512)
    pltpu.sync_copy(out_v, out_hbm.at[pl.ds(oo, _BG * _H)])


@jax.jit
def kernel(x, position_weight, level_weight):
    posh = jnp.concatenate([position_weight[:, :_H],
                            position_weight[:, _H:]], axis=0)
    lvlh = jnp.concatenate([level_weight[:, :_H],
                            level_weight[:, _H:]], axis=0)

    mesh = plsc.VectorSubcoreMesh(core_axis_name="c", subcore_axis_name="s")
    run = functools.partial(
        pl.kernel,
        mesh=mesh,
        out_type=jax.ShapeDtypeStruct((_B * _D,), jnp.float32),
        scratch_types=[
            pltpu.VMEM((16 * _H,), jnp.int32),
            pltpu.VMEM((32 * _H,), jnp.int32),
            pltpu.VMEM((_BG * _S,), jnp.float32),
            pltpu.VMEM((_S,), jnp.int32),
            pltpu.VMEM((_L * 16,), jnp.int32),
            pltpu.VMEM((_S * 16,), jnp.int32),
            pltpu.VMEM((16 * 16,), jnp.int32),
            pltpu.VMEM((32 * 16,), jnp.int32),
            pltpu.VMEM((_BG * _H,), jnp.float32),
            pltpu.VMEM_SHARED(((_L + _S) * 16,), jnp.int32),
            pltpu.SMEM((_S,), jnp.int32),
        ],
    )(_sc_body)
    out = run(x.reshape(_B * _S),
              lax.bitcast_convert_type(posh.reshape(_S * 2 * _H), jnp.int32),
              lax.bitcast_convert_type(lvlh.reshape(_L * 2 * _H), jnp.int32))
    return out.reshape(2, _B, _H).transpose(1, 0, 2).reshape(_B, _D)


# speedup vs baseline: 1.0009x; 1.0009x over previous
import functools
import jax
import jax.numpy as jnp
from jax import lax
from jax.experimental import pallas as pl
from jax.experimental.pallas import tpu as pltpu
from jax.experimental.pallas import tpu_sc as plsc

_B, _S, _D, _L = 128, 512, 1024, 256
_H = _D // 2
_BG = _B // 16


def _fa(a, b, c):
    t = a ^ b
    return t ^ c, (a & b) | (c & t)


def _ha(a, b):
    return a ^ b, a & b


def _count16(xs):
    c2 = []
    n1 = []
    for i in range(5):
        s, c = _fa(xs[3 * i], xs[3 * i + 1], xs[3 * i + 2])
        n1.append(s)
        c2.append(c)
    n1.append(xs[15])
    n2 = []
    for i in range(2):
        s, c = _fa(n1[3 * i], n1[3 * i + 1], n1[3 * i + 2])
        n2.append(s)
        c2.append(c)
    w0, c = _ha(n2[0], n2[1])
    c2.append(c)
    c4 = []
    m1 = []
    for i in range(2):
        s, c = _fa(c2[3 * i], c2[3 * i + 1], c2[3 * i + 2])
        m1.append(s)
        c4.append(c)
    m1.append(c2[6])
    m1.append(c2[7])
    s, c = _fa(m1[0], m1[1], m1[2])
    c4.append(c)
    w1, c = _ha(s, m1[3])
    c4.append(c)
    c8 = []
    s, c = _fa(c4[0], c4[1], c4[2])
    c8.append(c)
    w2, c = _ha(s, c4[3])
    c8.append(c)
    w3, w4 = _ha(c8[0], c8[1])
    return (w0, w1, w2, w3, w4)


def _global_add(g, w):
    g = list(g)
    g[0], carry = _ha(g[0], w[0])
    for k in range(1, 5):
        t = g[k] ^ w[k]
        s = t ^ carry
        carry = (g[k] & w[k]) | (carry & t)
        g[k] = s
    for k in range(5, 10):
        g[k], carry = _ha(g[k], carry)
    return tuple(g)


def _sc_body(x_hbm, pos_hbm, lvl_hbm, out_hbm,
             pkl_v, pkp_v, x_v, idx_v, lvlp_v, posp_v, lvl_loc, pos_loc,
             out_v, shared_v, idx_s):
    half = lax.axis_index("c")
    sid = lax.axis_index("s")

    co = pl.multiple_of(half * _H, 128)
    pltpu.sync_copy(
        lvl_hbm.at[pl.ds(pl.multiple_of(sid * 16, 8), 16), pl.ds(co, _H)],
        pkl_v)
    pltpu.sync_copy(
        pos_hbm.at[pl.ds(pl.multiple_of(sid * 32, 8), 32), pl.ds(co, _H)],
        pkp_v)

    def pack_lvl(r, c_):
        word = jnp.zeros((16,), jnp.int32)
        for bb in range(32):
            vi = pkl_v[r, pl.ds(bb * 16, 16)]
            bit = lax.shift_right_logical(vi, 31)
            word = word | (bit << bb)
        lvl_loc[pl.ds(pl.multiple_of(r * 16, 16), 16)] = word
        return c_

    lax.fori_loop(0, 16, pack_lvl, 0)

    def pack_pos(r, c_):
        word = jnp.zeros((16,), jnp.int32)
        for bb in range(32):
            vi = pkp_v[r, pl.ds(bb * 16, 16)]
            bit = lax.shift_right_logical(vi, 31) ^ 1
            word = word | (bit << bb)
        pos_loc[pl.ds(pl.multiple_of(r * 16, 16), 16)] = word
        return c_

    lax.fori_loop(0, 32, pack_pos, 0)

    pltpu.sync_copy(lvl_loc, shared_v.at[pl.ds(pl.multiple_of(sid * 256, 256), 256)])
    pltpu.sync_copy(pos_loc, shared_v.at[pl.ds(pl.multiple_of(_L * 16 + sid * 512, 512), 512)])
    plsc.subcore_barrier()
    pltpu.sync_copy(shared_v.at[pl.ds(0, _L * 16)], lvlp_v)
    pltpu.sync_copy(shared_v.at[pl.ds(_L * 16, _S * 16)], posp_v)

    pltpu.sync_copy(x_hbm.at[pl.ds(pl.multiple_of(sid * _BG * _S, 512), _BG * _S)], x_v)

    def b_step(b, carry):
        def q_step(j, c_):
            v = x_v[pl.ds(pl.multiple_of(b * _S + j * 16, 16), 16)]
            q = v * jnp.float32(_L - 1) + jnp.float32(0.5)
            qi = jnp.clip(q.astype(jnp.int32), 0, _L - 1) * 16
            idx_v[pl.ds(j * 16, 16)] = qi
            return c_

        lax.fori_loop(0, _S // 16, q_step, 0, unroll=4)

        z = jnp.zeros((16,), jnp.int32)

        @plsc.parallel_loop(0, _S // 16, carry=(z,) * 10)
        def s_loop(t, g):
            base = t * 16
            ivec = idx_v[pl.ds(base, 16)]
            xs = []
            for j in range(16):
                io = pl.multiple_of(ivec[j], 16)
                po = pl.multiple_of((base + j) * 16, 16)
                xs.append(lvlp_v[pl.ds(io, 16)] ^ posp_v[pl.ds(po, 16)])
            return _global_add(g, _count16(xs))

        g = s_loop

        low = g[0]
        for k in range(1, 8):
            low = low | g[k]
        ge = g[9] | (g[8] & low)
        for bb in range(32):
            bit = lax.shift_right_logical(ge, bb) & 1
            valf = jnp.where(bit == 1, jnp.float32(1.0), jnp.float32(-1.0))
            out_v[pl.ds(pl.multiple_of(b * _H + bb * 16, 16), 16)] = valf
        return carry

    lax.fori_loop(0, _BG, b_step, 0)

    oo = pl.multiple_of((half * 16 + sid) * (_BG * _H), 512)
    pltpu.sync_copy(out_v, out_hbm.at[pl.ds(oo, _BG * _H)])


@jax.jit
def kernel(x, position_weight, level_weight):
    mesh = plsc.VectorSubcoreMesh(core_axis_name="c", subcore_axis_name="s")
    run = functools.partial(
        pl.kernel,
        mesh=mesh,
        out_type=jax.ShapeDtypeStruct((_B * _D,), jnp.float32),
        scratch_types=[
            pltpu.VMEM((16, _H), jnp.int32),
            pltpu.VMEM((32, _H), jnp.int32),
            pltpu.VMEM((_BG * _S,), jnp.float32),
            pltpu.VMEM((_S,), jnp.int32),
            pltpu.VMEM((_L * 16,), jnp.int32),
            pltpu.VMEM((_S * 16,), jnp.int32),
            pltpu.VMEM((16 * 16,), jnp.int32),
            pltpu.VMEM((32 * 16,), jnp.int32),
            pltpu.VMEM((_BG * _H,), jnp.float32),
            pltpu.VMEM_SHARED(((_L + _S) * 16,), jnp.int32),
            pltpu.SMEM((_S,), jnp.int32),
        ],
    )(_sc_body)
    out = run(x.reshape(_B * _S),
              lax.bitcast_convert_type(position_weight, jnp.int32),
              lax.bitcast_convert_type(level_weight, jnp.int32))
    return out.reshape(2, _B, _H).transpose(1, 0, 2).reshape(_B, _D)


# speedup vs baseline: 1.0145x; 1.0135x over previous
import functools
import jax
import jax.numpy as jnp
from jax import lax
from jax.experimental import pallas as pl
from jax.experimental.pallas import tpu as pltpu
from jax.experimental.pallas import tpu_sc as plsc

_B, _S, _D, _L = 128, 512, 1024, 256
_H = _D // 2
_BG = _B // 16


def _fa(a, b, c):
    t = a ^ b
    return t ^ c, (a & b) | (c & t)


def _ha(a, b):
    return a ^ b, a & b


def _count16(xs):
    c2 = []
    n1 = []
    for i in range(5):
        s, c = _fa(xs[3 * i], xs[3 * i + 1], xs[3 * i + 2])
        n1.append(s)
        c2.append(c)
    n1.append(xs[15])
    n2 = []
    for i in range(2):
        s, c = _fa(n1[3 * i], n1[3 * i + 1], n1[3 * i + 2])
        n2.append(s)
        c2.append(c)
    w0, c = _ha(n2[0], n2[1])
    c2.append(c)
    c4 = []
    m1 = []
    for i in range(2):
        s, c = _fa(c2[3 * i], c2[3 * i + 1], c2[3 * i + 2])
        m1.append(s)
        c4.append(c)
    m1.append(c2[6])
    m1.append(c2[7])
    s, c = _fa(m1[0], m1[1], m1[2])
    c4.append(c)
    w1, c = _ha(s, m1[3])
    c4.append(c)
    c8 = []
    s, c = _fa(c4[0], c4[1], c4[2])
    c8.append(c)
    w2, c = _ha(s, c4[3])
    c8.append(c)
    w3, w4 = _ha(c8[0], c8[1])
    return (w0, w1, w2, w3, w4)


def _global_add(g, w):
    g = list(g)
    g[0], carry = _ha(g[0], w[0])
    for k in range(1, 5):
        t = g[k] ^ w[k]
        s = t ^ carry
        carry = (g[k] & w[k]) | (carry & t)
        g[k] = s
    for k in range(5, 10):
        g[k], carry = _ha(g[k], carry)
    return tuple(g)


def _sc_body(x_hbm, pos_hbm, lvl_hbm, out_hbm,
             pkl_v, pkp_v, x_v, lvlp_v, posp_v, lvl_loc, pos_loc,
             out_v, shared_v):
    half = lax.axis_index("c")
    sid = lax.axis_index("s")

    co = pl.multiple_of(half * _H, 128)
    pltpu.sync_copy(
        lvl_hbm.at[pl.ds(pl.multiple_of(sid * 16, 8), 16), pl.ds(co, _H)],
        pkl_v)
    pltpu.sync_copy(
        pos_hbm.at[pl.ds(pl.multiple_of(sid * 32, 8), 32), pl.ds(co, _H)],
        pkp_v)

    half_f = jnp.full((16,), 0.5, jnp.float32)

    def pack_lvl(r, c_):
        word = jnp.zeros((16,), jnp.int32)
        for bb in range(32):
            v = pkl_v[r, pl.ds(bb * 16, 16)]
            bit = (half_f - half_f * lax.sign(v)).astype(jnp.int32)
            word = word | (bit << bb)
        lvl_loc[pl.ds(pl.multiple_of(r * 16, 16), 16)] = word
        return c_

    lax.fori_loop(0, 16, pack_lvl, 0)

    def pack_pos(r, c_):
        word = jnp.zeros((16,), jnp.int32)
        for bb in range(32):
            v = pkp_v[r, pl.ds(bb * 16, 16)]
            bit = (half_f + half_f * lax.sign(v)).astype(jnp.int32)
            word = word | (bit << bb)
        pos_loc[pl.ds(pl.multiple_of(r * 16, 16), 16)] = word
        return c_

    lax.fori_loop(0, 32, pack_pos, 0)

    pltpu.sync_copy(lvl_loc, shared_v.at[pl.ds(pl.multiple_of(sid * 256, 256), 256)])
    pltpu.sync_copy(pos_loc, shared_v.at[pl.ds(pl.multiple_of(_L * 16 + sid * 512, 512), 512)])
    plsc.subcore_barrier()
    pltpu.sync_copy(shared_v.at[pl.ds(0, _L * 16)], lvlp_v)
    pltpu.sync_copy(shared_v.at[pl.ds(_L * 16, _S * 16)], posp_v)

    pltpu.sync_copy(x_hbm.at[pl.ds(pl.multiple_of(sid * _BG * _S, 512), _BG * _S)], x_v)

    def b_step(b, carry):
        z = jnp.zeros((16,), jnp.int32)

        @plsc.parallel_loop(0, _S // 16, carry=(z,) * 10)
        def s_loop(t, g):
            base = t * 16
            v = x_v[pl.ds(pl.multiple_of(b * _S + base, 16), 16)]
            q = v * jnp.float32(_L - 1) + jnp.float32(0.5)
            ivec = jnp.clip(q.astype(jnp.int32), 0, _L - 1) * 16
            xs = []
            for j in range(16):
                io = pl.multiple_of(ivec[j], 16)
                po = pl.multiple_of((base + j) * 16, 16)
                xs.append(lvlp_v[pl.ds(io, 16)] ^ posp_v[pl.ds(po, 16)])
            return _global_add(g, _count16(xs))

        g = s_loop

        low = g[0]
        for k in range(1, 8):
            low = low | g[k]
        ge = g[9] | (g[8] & low)
        for bb in range(32):
            bit = lax.shift_right_logical(ge, bb) & 1
            valf = jnp.where(bit == 1, jnp.float32(1.0), jnp.float32(-1.0))
            out_v[pl.ds(pl.multiple_of(b * _H + bb * 16, 16), 16)] = valf
        return carry

    lax.fori_loop(0, _BG, b_step, 0)

    oo = pl.multiple_of((half * 16 + sid) * (_BG * _H), 512)
    pltpu.sync_copy(out_v, out_hbm.at[pl.ds(oo, _BG * _H)])


@jax.jit
def kernel(x, position_weight, level_weight):
    mesh = plsc.VectorSubcoreMesh(core_axis_name="c", subcore_axis_name="s")
    run = functools.partial(
        pl.kernel,
        mesh=mesh,
        out_type=jax.ShapeDtypeStruct((_B * _D,), jnp.float32),
        scratch_types=[
            pltpu.VMEM((16, _H), jnp.float32),
            pltpu.VMEM((32, _H), jnp.float32),
            pltpu.VMEM((_BG * _S,), jnp.float32),
            pltpu.VMEM((_L * 16,), jnp.int32),
            pltpu.VMEM((_S * 16,), jnp.int32),
            pltpu.VMEM((16 * 16,), jnp.int32),
            pltpu.VMEM((32 * 16,), jnp.int32),
            pltpu.VMEM((_BG * _H,), jnp.float32),
            pltpu.VMEM_SHARED(((_L + _S) * 16,), jnp.int32),
        ],
    )(_sc_body)
    out = run(x.reshape(_B * _S), position_weight, level_weight)
    return out.reshape(2, _B, _H).transpose(1, 0, 2).reshape(_B, _D)
